# Initial kernel scaffold; baseline (speedup 1.0000x reference)
import jax
import jax.numpy as jnp
from jax.experimental import pallas as pl


def kernel(node_features, edge_index, W1, b1, W2, b2):
    raise NotImplementedError("write your pallas kernel here")


# speedup vs baseline: 1.2610x; 1.2610x over previous
import functools

import jax
import jax.numpy as jnp
from jax import lax
from jax.experimental import pallas as pl
from jax.experimental.pallas import tpu as pltpu
from jax.experimental.pallas import tpu_sc as plsc

N_NODES = 10000
N_EDGES = 320000
D = 128

NC = 2
NS = 16
NW = NC * NS
L = 16

EPW = N_EDGES // NW
G = 80
NCHUNK = EPW // G


def _proj_body(nf_ref, w1_ref, b1_ref, w2t_ref, b2_ref, a_ref, b_ref, wc_ref):
    x = nf_ref[...]
    w1a = w1_ref[0:D, :]
    w1b = w1_ref[D:2 * D, :]
    a_ref[...] = jnp.dot(x, w1a, preferred_element_type=jnp.float32) + b1_ref[...]
    b_ref[...] = jnp.dot(x, w1b, preferred_element_type=jnp.float32)
    w = w2t_ref[0, :] - w2t_ref[1, :]
    c = b2_ref[0, 0] - b2_ref[0, 1]
    wc_ref[...] = jnp.concatenate([w, jnp.full((L,), c, jnp.float32)]).reshape(1, D + L)


def _node_projections(nf, W1, b1, W2, b2):
    BN = 400
    grid = (N_NODES // BN,)
    return pl.pallas_call(
        _proj_body,
        grid=grid,
        in_specs=[
            pl.BlockSpec((BN, D), lambda i: (i, 0)),
            pl.BlockSpec((2 * D, D), lambda i: (0, 0)),
            pl.BlockSpec((1, D), lambda i: (0, 0)),
            pl.BlockSpec((2, D), lambda i: (0, 0)),
            pl.BlockSpec((1, 2), lambda i: (0, 0)),
        ],
        out_specs=[
            pl.BlockSpec((BN, D), lambda i: (i, 0)),
            pl.BlockSpec((BN, D), lambda i: (i, 0)),
            pl.BlockSpec((1, D + L), lambda i: (0, 0)),
        ],
        out_shape=[
            jax.ShapeDtypeStruct((N_NODES, D), jnp.float32),
            jax.ShapeDtypeStruct((N_NODES, D), jnp.float32),
            jax.ShapeDtypeStruct((1, D + L), jnp.float32),
        ],
    )(nf, W1, b1.reshape(1, D), W2.T, b2.reshape(1, 2))


def _edge_body(a_hbm, b_hbm, src_hbm, dst_hbm, wc_hbm, oc_hbm, ot_hbm,
               idx_s, idx_d, buf_a0, buf_b0, buf_a1, buf_b1,
               wc_v, oc_all, ot_all, sg0, sg1):
    wid = lax.axis_index("s") * NC + lax.axis_index("c")
    base = wid * EPW

    pltpu.sync_copy(wc_hbm, wc_v)
    pltpu.sync_copy(src_hbm.at[pl.ds(base, EPW)], idx_s)
    pltpu.sync_copy(dst_hbm.at[pl.ds(base, EPW)], idx_d)
    cvec = wc_v[pl.ds(D, L)]
    iota = lax.iota(jnp.int32, L)
    rows = [jnp.full((L,), g * L, jnp.int32) + iota for g in range(G // L)]
    slots = ((buf_a0, buf_b0, sg0), (buf_a1, buf_b1, sg1))

    def fire(c, s):
        ba, bb, sem = slots[s]
        off = c * G
        pltpu.async_copy(a_hbm.at[idx_s.at[pl.ds(off, G)]], ba, sem)
        pltpu.async_copy(b_hbm.at[idx_d.at[pl.ds(off, G)]], bb, sem)

    def drain(s):
        ba, bb, sem = slots[s]
        pltpu.make_async_copy(a_hbm.at[idx_s.at[pl.ds(0, G)]], ba, sem).wait()
        pltpu.make_async_copy(b_hbm.at[idx_d.at[pl.ds(0, G)]], bb, sem).wait()

    def compute(c, s):
        ba, bb, _ = slots[s]

        def kstep(k, accs):
            colk = jnp.full((L,), 0, jnp.int32) + k
            wk = plsc.load_gather(wc_v, [colk])
            new = []
            for g in range(G // L):
                xa = plsc.load_gather(ba, [rows[g], colk])
                xb = plsc.load_gather(bb, [rows[g], colk])
                x = jnp.maximum(xa + xb, 0.0)
                new.append(accs[g] + x * wk)
            return tuple(new)

        accs = lax.fori_loop(0, D, kstep, tuple(cvec for _ in range(G // L)),
                             unroll=2)
        obase = c * G
        for g in range(G // L):
            d = accs[g]
            oc_all[pl.ds(obase + g * L, L)] = 1.0 / (1.0 + jnp.exp(-d))
            ot_all[pl.ds(obase + g * L, L)] = 1.0 / (1.0 + jnp.exp(d))

    fire(0, 0)

    def pair(j, carry):
        c0 = 2 * j
        fire(c0 + 1, 1)
        drain(0)
        compute(c0, 0)
        fire(c0 + 2, 0)
        drain(1)
        compute(c0 + 1, 1)
        return carry

    lax.fori_loop(0, (NCHUNK - 1) // 2, pair, 0)
    drain(0)
    compute(NCHUNK - 1, 0)
    pltpu.sync_copy(oc_all, oc_hbm.at[pl.ds(base, EPW)])
    pltpu.sync_copy(ot_all, ot_hbm.at[pl.ds(base, EPW)])


def _edge_scores(A, B, src, dst, wc):
    mesh = plsc.VectorSubcoreMesh(core_axis_name="c", subcore_axis_name="s")
    f = functools.partial(
        pl.kernel,
        mesh=mesh,
        out_type=[
            jax.ShapeDtypeStruct((N_EDGES,), jnp.float32),
            jax.ShapeDtypeStruct((N_EDGES,), jnp.float32),
        ],
        scratch_types=[
            pltpu.VMEM((EPW,), jnp.int32),
            pltpu.VMEM((EPW,), jnp.int32),
            pltpu.VMEM((G, D), jnp.float32),
            pltpu.VMEM((G, D), jnp.float32),
            pltpu.VMEM((G, D), jnp.float32),
            pltpu.VMEM((G, D), jnp.float32),
            pltpu.VMEM((D + L,), jnp.float32),
            pltpu.VMEM((EPW,), jnp.float32),
            pltpu.VMEM((EPW,), jnp.float32),
            pltpu.SemaphoreType.DMA,
            pltpu.SemaphoreType.DMA,
        ],
        compiler_params=pltpu.CompilerParams(needs_layout_passes=False),
    )(_edge_body)
    return f(A, B, src, dst, wc)


def kernel(node_features, edge_index, W1, b1, W2, b2):
    src = edge_index[0].astype(jnp.int32)
    dst = edge_index[1].astype(jnp.int32)
    A, B, wc = _node_projections(node_features, W1, b1, W2, b2)
    edge_c, edge_t = _edge_scores(A, B, src, dst, wc.reshape(D + L))
    return (edge_c, edge_t)


# speedup vs baseline: 8.3095x; 6.5894x over previous
import functools

import jax
import jax.numpy as jnp
from jax import lax
from jax.experimental import pallas as pl
from jax.experimental.pallas import tpu as pltpu
from jax.experimental.pallas import tpu_sc as plsc

N_NODES = 10000
N_EDGES = 320000
D = 128

NC = 2
NS = 16
NW = NC * NS
L = 16

EPW = N_EDGES // NW
G = 80
NCHUNK = EPW // G


def _proj_body(nf_ref, w1_ref, b1_ref, w2t_ref, b2_ref, a_ref, b_ref, wc_ref):
    x = nf_ref[...]
    w1a = w1_ref[0:D, :]
    w1b = w1_ref[D:2 * D, :]
    a_ref[...] = jnp.dot(x, w1a, preferred_element_type=jnp.float32) + b1_ref[...]
    b_ref[...] = jnp.dot(x, w1b, preferred_element_type=jnp.float32)
    w = w2t_ref[0, :] - w2t_ref[1, :]
    c = b2_ref[0, 0] - b2_ref[0, 1]
    wc_ref[...] = jnp.concatenate([w, jnp.full((L,), c, jnp.float32)]).reshape(1, D + L)


def _node_projections(nf, W1, b1, W2, b2):
    BN = 400
    grid = (N_NODES // BN,)
    return pl.pallas_call(
        _proj_body,
        grid=grid,
        in_specs=[
            pl.BlockSpec((BN, D), lambda i: (i, 0)),
            pl.BlockSpec((2 * D, D), lambda i: (0, 0)),
            pl.BlockSpec((1, D), lambda i: (0, 0)),
            pl.BlockSpec((2, D), lambda i: (0, 0)),
            pl.BlockSpec((1, 2), lambda i: (0, 0)),
        ],
        out_specs=[
            pl.BlockSpec((BN, D), lambda i: (i, 0)),
            pl.BlockSpec((BN, D), lambda i: (i, 0)),
            pl.BlockSpec((1, D + L), lambda i: (0, 0)),
        ],
        out_shape=[
            jax.ShapeDtypeStruct((N_NODES, D), jnp.float32),
            jax.ShapeDtypeStruct((N_NODES, D), jnp.float32),
            jax.ShapeDtypeStruct((1, D + L), jnp.float32),
        ],
    )(nf, W1, b1.reshape(1, D), W2.T, b2.reshape(1, 2))


def _edge_body(a_hbm, b_hbm, src_hbm, dst_hbm, wc_hbm, oc_hbm, ot_hbm,
               idx_s, idx_d, buf_a0, buf_b0, buf_a1, buf_b1,
               wc_v, oc_all, ot_all, sg0, sg1):
    wid = lax.axis_index("s") * NC + lax.axis_index("c")
    base = wid * EPW

    pltpu.sync_copy(wc_hbm, wc_v)
    pltpu.sync_copy(src_hbm.at[pl.ds(base, EPW)], idx_s)
    pltpu.sync_copy(dst_hbm.at[pl.ds(base, EPW)], idx_d)
    cvec = wc_v[pl.ds(D, L)]
    iota = lax.iota(jnp.int32, L)
    rows = [jnp.full((L,), g * L, jnp.int32) + iota for g in range(G // L)]
    slots = ((buf_a0, buf_b0, sg0), (buf_a1, buf_b1, sg1))

    def fire(c, s):
        ba, bb, sem = slots[s]
        off = c * G
        pltpu.async_copy(a_hbm.at[idx_s.at[pl.ds(off, G)]], ba, sem)
        pltpu.async_copy(b_hbm.at[idx_d.at[pl.ds(off, G)]], bb, sem)

    def drain(s):
        ba, bb, sem = slots[s]
        pltpu.make_async_copy(a_hbm.at[idx_s.at[pl.ds(0, G)]], ba, sem).wait()
        pltpu.make_async_copy(b_hbm.at[idx_d.at[pl.ds(0, G)]], bb, sem).wait()

    def compute(c, s):
        ba, bb, _ = slots[s]

        def kstep(k, accs):
            colk = (iota + k) & (D - 1)
            wk = plsc.load_gather(wc_v, [colk])
            new = []
            for g in range(G // L):
                xa = plsc.load_gather(ba, [rows[g], colk])
                xb = plsc.load_gather(bb, [rows[g], colk])
                x = jnp.maximum(xa + xb, 0.0)
                new.append(accs[g] + x * wk)
            return tuple(new)

        accs = lax.fori_loop(0, D, kstep, tuple(cvec for _ in range(G // L)),
                             unroll=2)
        obase = c * G
        for g in range(G // L):
            d = accs[g]
            oc_all[pl.ds(obase + g * L, L)] = 1.0 / (1.0 + jnp.exp(-d))
            ot_all[pl.ds(obase + g * L, L)] = 1.0 / (1.0 + jnp.exp(d))

    fire(0, 0)

    def pair(j, carry):
        c0 = 2 * j
        fire(c0 + 1, 1)
        drain(0)
        compute(c0, 0)
        fire(c0 + 2, 0)
        drain(1)
        compute(c0 + 1, 1)
        return carry

    lax.fori_loop(0, (NCHUNK - 1) // 2, pair, 0)
    drain(0)
    compute(NCHUNK - 1, 0)
    pltpu.sync_copy(oc_all, oc_hbm.at[pl.ds(base, EPW)])
    pltpu.sync_copy(ot_all, ot_hbm.at[pl.ds(base, EPW)])


def _edge_scores(A, B, src, dst, wc):
    mesh = plsc.VectorSubcoreMesh(core_axis_name="c", subcore_axis_name="s")
    f = functools.partial(
        pl.kernel,
        mesh=mesh,
        out_type=[
            jax.ShapeDtypeStruct((N_EDGES,), jnp.float32),
            jax.ShapeDtypeStruct((N_EDGES,), jnp.float32),
        ],
        scratch_types=[
            pltpu.VMEM((EPW,), jnp.int32),
            pltpu.VMEM((EPW,), jnp.int32),
            pltpu.VMEM((G, D), jnp.float32),
            pltpu.VMEM((G, D), jnp.float32),
            pltpu.VMEM((G, D), jnp.float32),
            pltpu.VMEM((G, D), jnp.float32),
            pltpu.VMEM((D + L,), jnp.float32),
            pltpu.VMEM((EPW,), jnp.float32),
            pltpu.VMEM((EPW,), jnp.float32),
            pltpu.SemaphoreType.DMA,
            pltpu.SemaphoreType.DMA,
        ],
        compiler_params=pltpu.CompilerParams(needs_layout_passes=False),
    )(_edge_body)
    return f(A, B, src, dst, wc)


def kernel(node_features, edge_index, W1, b1, W2, b2):
    src = edge_index[0].astype(jnp.int32)
    dst = edge_index[1].astype(jnp.int32)
    A, B, wc = _node_projections(node_features, W1, b1, W2, b2)
    edge_c, edge_t = _edge_scores(A, B, src, dst, wc.reshape(D + L))
    return (edge_c, edge_t)
